# Initial kernel scaffold; baseline (speedup 1.0000x reference)
import jax
import jax.numpy as jnp
from jax.experimental import pallas as pl


def kernel(x, Wr, br, W1, b1, W2, b2, scale):
    raise NotImplementedError("write your pallas kernel here")


# speedup vs baseline: 2.6652x; 2.6652x over previous
import jax
import jax.numpy as jnp
from jax.experimental import pallas as pl
from jax.experimental.pallas import tpu as pltpu


def _moe_body(tok_ref, wr_ref, br_ref, w1_ref, b1_ref, w2_ref, b2_ref,
              scale_ref, out_ref, eidx_ref):
    e = pl.program_id(0)
    n, c = tok_ref.shape
    n_experts = wr_ref.shape[0]

    @pl.when(e == 0)
    def _():
        logits = jax.lax.dot_general(
            tok_ref[:], wr_ref[:], (((1,), (1,)), ((), ())),
            preferred_element_type=jnp.float32) + br_ref[:]
        maxv = jnp.max(logits, axis=1, keepdims=True)
        lane = jax.lax.broadcasted_iota(jnp.int32, (n, n_experts), 1)
        eidx_ref[:] = jnp.min(
            jnp.where(logits >= maxv, lane, n_experts), axis=1, keepdims=True)
        out_ref[:] = tok_ref[:]

    h1 = jax.lax.dot_general(
        tok_ref[:], w1_ref[0], (((1,), (1,)), ((), ())),
        preferred_element_type=jnp.float32) + b1_ref[0]
    h1 = 0.5 * h1 * (1.0 + jax.lax.erf(h1 * 0.7071067811865476))
    ye = jax.lax.dot_general(
        h1, w2_ref[0], (((1,), (1,)), ((), ())),
        preferred_element_type=jnp.float32) + b2_ref[0]
    mask = (eidx_ref[:] == e).astype(jnp.float32)
    out_ref[:] += (scale_ref[0, 0] * mask) * ye


def kernel(x, Wr, br, W1, b1, W2, b2, scale):
    b, c, h, w = x.shape
    n = b * h * w
    E, HID, _ = W1.shape
    tokens = jnp.transpose(x, (0, 2, 3, 1)).reshape(n, c)

    out = pl.pallas_call(
        _moe_body,
        grid=(E,),
        in_specs=[
            pl.BlockSpec((n, c), lambda e: (0, 0)),
            pl.BlockSpec((E, c), lambda e: (0, 0)),
            pl.BlockSpec((1, E), lambda e: (0, 0)),
            pl.BlockSpec((1, HID, c), lambda e: (e, 0, 0)),
            pl.BlockSpec((1, 1, HID), lambda e: (e, 0, 0)),
            pl.BlockSpec((1, c, HID), lambda e: (e, 0, 0)),
            pl.BlockSpec((1, 1, c), lambda e: (e, 0, 0)),
            pl.BlockSpec((1, 1), lambda e: (0, 0)),
        ],
        out_specs=pl.BlockSpec((n, c), lambda e: (0, 0)),
        out_shape=jax.ShapeDtypeStruct((n, c), jnp.float32),
        scratch_shapes=[pltpu.VMEM((n, 1), jnp.int32)],
    )(tokens, Wr, br.reshape(1, E), W1, b1.reshape(E, 1, HID),
      W2, b2.reshape(E, 1, c), scale.reshape(1, 1))

    return jnp.transpose(out.reshape(b, h, w, c), (0, 3, 1, 2))
